# Initial kernel scaffold; baseline (speedup 1.0000x reference)
import jax
import jax.numpy as jnp
from jax.experimental import pallas as pl


def kernel(z, edge_index):
    raise NotImplementedError("write your pallas kernel here")


# speedup vs baseline: 1.3414x; 1.3414x over previous
import functools

import jax
import jax.numpy as jnp
from jax import lax
from jax.experimental import pallas as pl
from jax.experimental.pallas import tpu as pltpu
from jax.experimental.pallas import tpu_sc as plsc

_LANES = 16


@functools.lru_cache(maxsize=None)
def _make_decoder(n_nodes: int, d: int, n_edges: int):
    info = plsc.get_sparse_core_info()
    nw = info.num_cores * info.num_subcores
    assert d % _LANES == 0
    assert n_edges % nw == 0
    per_w = n_edges // nw
    chunk = 0
    for c in range(128, 15, -16):
        if per_w % c == 0:
            chunk = c
            break
    assert chunk > 0
    n_chunks = per_w // chunk
    kd = d // _LANES

    mesh = plsc.VectorSubcoreMesh(core_axis_name="c", subcore_axis_name="s")

    @functools.partial(
        pl.kernel,
        out_type=jax.ShapeDtypeStruct((n_edges,), jnp.float32),
        mesh=mesh,
        compiler_params=pltpu.CompilerParams(needs_layout_passes=False),
        scratch_types=[
            pltpu.VMEM((n_chunks, chunk), jnp.int32),
            pltpu.VMEM((n_chunks, chunk), jnp.int32),
            pltpu.VMEM((2, chunk, d), jnp.float32),
            pltpu.VMEM((2, chunk, d), jnp.float32),
            pltpu.VMEM((per_w,), jnp.float32),
            pltpu.SemaphoreType.DMA,
            pltpu.SemaphoreType.DMA,
        ],
    )
    def decode(z_hbm, ei_hbm, out_hbm, idx_s, idx_d, src_buf, dst_buf,
               out_buf, sem0, sem1):
        wid = lax.axis_index("s") * info.num_cores + lax.axis_index("c")
        base = wid * per_w
        sems = (sem0, sem1)

        pltpu.sync_copy(ei_hbm.at[0, wid], idx_s)
        pltpu.sync_copy(ei_hbm.at[1, wid], idx_d)

        def fire(c, slot):
            pltpu.make_async_copy(
                z_hbm.at[idx_s.at[c]], src_buf.at[slot], sems[slot]).start()
            pltpu.make_async_copy(
                z_hbm.at[idx_d.at[c]], dst_buf.at[slot], sems[slot]).start()

        def drain(c, slot):
            pltpu.make_async_copy(
                z_hbm.at[idx_s.at[c]], src_buf.at[slot], sems[slot]).wait()
            pltpu.make_async_copy(
                z_hbm.at[idx_d.at[c]], dst_buf.at[slot], sems[slot]).wait()

        lane = lax.iota(jnp.int32, 16)

        def compute(c, slot):
            out_base = c * chunk

            def group_body(g, _):
                e_vec = g * _LANES + lane

                def kstep(k, acc):
                    kv = jnp.full((_LANES,), k, jnp.int32)
                    s = plsc.load_gather(src_buf.at[slot], [e_vec, kv])
                    t = plsc.load_gather(dst_buf.at[slot], [e_vec, kv])
                    return acc + s * t

                acc = lax.fori_loop(0, d, kstep,
                                    jnp.zeros((_LANES,), jnp.float32),
                                    unroll=8)
                res = 1.0 / (1.0 + jnp.exp(-acc))
                out_buf[pl.ds(out_base + g * _LANES, _LANES)] = res
                return 0

            lax.fori_loop(0, chunk // _LANES, group_body, 0)

        fire(0, 0)

        def pipe_body(i, _):
            c = 2 * i
            fire(c + 1, 1)
            drain(c, 0)
            compute(c, 0)

            if n_chunks % 2 == 0:
                @pl.when(c + 2 < n_chunks)
                def _():
                    fire(c + 2, 0)
            else:
                fire(c + 2, 0)

            drain(c + 1, 1)
            compute(c + 1, 1)
            return 0

        lax.fori_loop(0, n_chunks // 2, pipe_body, 0)
        if n_chunks % 2 == 1:
            drain(n_chunks - 1, 0)
            compute(n_chunks - 1, 0)

        pltpu.sync_copy(out_buf, out_hbm.at[pl.ds(base, per_w)])

    return decode, nw, n_chunks, chunk


def kernel(z, edge_index):
    n_nodes, d = z.shape
    n_edges = edge_index.shape[1]
    decode, nw, n_chunks, chunk = _make_decoder(n_nodes, d, n_edges)
    ei = edge_index.astype(jnp.int32).reshape(2, nw, n_chunks, chunk)
    return decode(z, ei)
